# baseline (speedup 1.0000x reference)
import functools

import jax
import jax.numpy as jnp
from jax import lax
from jax.experimental import pallas as pl
from jax.experimental.pallas import tpu as pltpu
from jax.experimental.pallas import tpu_sc as plsc

N = 10000
D = 128
DH = D // 2
E = 320000

NC = 2
NT = 16
WIN = 128
NWIN = 159
E_PAD = NT * NWIN * WIN
NROW = 10112
CROWS = NROW // NT


def _sc_mesh_kernel():
    mesh = plsc.VectorSubcoreMesh(core_axis_name="c", subcore_axis_name="s")
    out = jax.ShapeDtypeStruct((NC, NROW, DH), jnp.float32)

    @functools.partial(
        pl.kernel,
        mesh=mesh,
        out_type=[out] * 8,
        compiler_params=pltpu.CompilerParams(use_tc_tiling_on_sc=False),
        scratch_types=[
            pltpu.VMEM((NWIN, WIN), jnp.int32),
            pltpu.VMEM((NWIN, WIN), jnp.int32),
            pltpu.VMEM((NWIN, WIN), jnp.float32),
            pltpu.VMEM((3, WIN, DH), jnp.float32),
            pltpu.VMEM_SHARED((NROW, DH), jnp.float32),
            pltpu.SemaphoreType.DMA,
            pltpu.SemaphoreType.DMA,
            pltpu.SemaphoreType.DMA,
            pltpu.SemaphoreType.DMA,
            pltpu.SemaphoreType.DMA,
            pltpu.SemaphoreType.DMA,
        ],
    )
    def sc_spmm(g00, g01, g10, g11, src_h, dst_h, norm_h, zeros_h,
                o100, o101, o110, o111, o200, o201, o210, o211,
                src_v, dst_v, norm_v, rows3, acc,
                sga, sgb, sgc, ssa, ssb, ssc):
        c = lax.axis_index("c")
        s = lax.axis_index("s")
        pltpu.sync_copy(src_h.at[s], src_v)
        pltpu.sync_copy(dst_h.at[s], dst_v)
        pltpu.sync_copy(norm_h.at[c].at[s], norm_v)
        o1 = ((o100, o101), (o110, o111))
        o2 = ((o200, o201), (o210, o211))

        bufs = (rows3.at[0], rows3.at[1], rows3.at[2])
        sgs = (sga, sgb, sgc)
        sss = (ssa, ssb, ssc)

        def scale(buf, w):
            @plsc.parallel_loop(0, WIN // 16, unroll=2)
            def _grp(g):
                nvec = norm_v[w, pl.ds(g * 16, 16)]
                for k in range(16):
                    nv = nvec[k]
                    r = g * 16 + k
                    for j in range(DH // 16):
                        sl = pl.ds(j * 16, 16)
                        buf[r, sl] = buf[r, sl] * nv

        for lvl in range(2):
            for p in range(2):
                for h in range(2):
                    oref = (o1 if lvl == 0 else o2)[p][h]
                    if lvl == 0:
                        gt = ((g00, g01), (g10, g11))[p][h]
                    else:
                        gt = o1[p][h].at[c]
                    pltpu.sync_copy(zeros_h.at[pl.ds(s * CROWS, CROWS)],
                                    acc.at[pl.ds(s * CROWS, CROWS)])
                    plsc.subcore_barrier()

                    for b in range(2):
                        pltpu.make_async_copy(gt.at[src_v.at[b]], bufs[b],
                                              sgs[b]).start()

                    def tri_body(t, _):
                        w0 = 3 * t
                        for b in range(3):
                            w = w0 + b
                            nb = (b + 2) % 3
                            pltpu.make_async_copy(gt.at[src_v.at[w]],
                                                  bufs[b], sgs[b]).wait()
                            scale(bufs[b], w)
                            pltpu.async_copy(bufs[b], acc.at[dst_v.at[w]],
                                             sss[b], add=True)

                            @pl.when((w >= 1) & (w + 2 < NWIN))
                            def _():
                                pltpu.make_async_copy(
                                    bufs[nb], acc.at[dst_v.at[w - 1]],
                                    sss[nb]).wait()

                            @pl.when(w + 2 < NWIN)
                            def _():
                                pltpu.make_async_copy(
                                    gt.at[src_v.at[w + 2]], bufs[nb],
                                    sgs[nb]).start()
                        return 0

                    lax.fori_loop(0, NWIN // 3, tri_body, 0, unroll=False)
                    for b in range(3):
                        pltpu.make_async_copy(
                            bufs[b], acc.at[dst_v.at[NWIN - 3 + b]],
                            sss[b]).wait()
                    plsc.subcore_barrier()
                    pltpu.sync_copy(acc.at[pl.ds(s * CROWS, CROWS)],
                                    oref.at[c].at[pl.ds(s * CROWS, CROWS)])
                    plsc.subcore_barrier()

    return sc_spmm


_BLK = 1000


def _tc_head(xr, xi, a00, a01, a10, a11, b00, b01, b10, b11, w, b,
             out_r, out_i):
    f32 = jnp.float32
    w02 = w[0] - w[2]
    u_m_v = jnp.concatenate([a00[0] - a10[1], a01[0] - a11[1]], axis=1)
    p_m_q = jnp.concatenate([b00[0] - b10[1], b01[0] - b11[1]], axis=1)
    u_p_v = jnp.concatenate([a10[0] + a00[1], a11[0] + a01[1]], axis=1)
    p_p_q = jnp.concatenate([b10[0] + b00[1], b11[0] + b01[1]], axis=1)
    or_ = jnp.dot(xr[...], w02, preferred_element_type=f32)
    or_ += jnp.dot(u_m_v, w[1], preferred_element_type=f32)
    or_ += 2.0 * jnp.dot(p_m_q, w[2], preferred_element_type=f32)
    oi_ = jnp.dot(xi[...], w02, preferred_element_type=f32)
    oi_ += jnp.dot(u_p_v, w[1], preferred_element_type=f32)
    oi_ += 2.0 * jnp.dot(p_p_q, w[2], preferred_element_type=f32)
    out_r[...] = or_ + b[...]
    out_i[...] = oi_ + b[...]


def kernel(x_real, x_imag, edge_index, norm_real, norm_imag, weight, bias):
    pad = E_PAD - E
    src = jnp.concatenate([edge_index[1], jnp.zeros((pad,), jnp.int32)])
    dst = jnp.concatenate(
        [edge_index[0], N + (jnp.arange(pad, dtype=jnp.int32) % 16)])
    nr = jnp.concatenate([norm_real, jnp.zeros((pad,), jnp.float32)])
    ni = jnp.concatenate([norm_imag, jnp.zeros((pad,), jnp.float32)])

    src3 = src.reshape(NT, NWIN, WIN)
    dst3 = dst.reshape(NT, NWIN, WIN)
    norms = jnp.stack([nr, ni]).reshape(NC, NT, NWIN, WIN)
    g00, g01 = x_real[:, :DH], x_real[:, DH:]
    g10, g11 = x_imag[:, :DH], x_imag[:, DH:]
    zeros = jnp.zeros((NROW, DH), jnp.float32)

    outs = _sc_mesh_kernel()(g00, g01, g10, g11, src3, dst3, norms, zeros)

    nb = pl.cdiv(N, _BLK)
    node_spec = pl.BlockSpec((_BLK, D), lambda i: (i, 0))
    pair_spec = pl.BlockSpec((NC, _BLK, DH), lambda i: (0, i, 0))
    out_r, out_i = pl.pallas_call(
        _tc_head,
        grid=(nb,),
        in_specs=[
            node_spec, node_spec,
            pair_spec, pair_spec, pair_spec, pair_spec,
            pair_spec, pair_spec, pair_spec, pair_spec,
            pl.BlockSpec((3, D, D), lambda i: (0, 0, 0)),
            pl.BlockSpec((1, D), lambda i: (0, 0)),
        ],
        out_specs=[node_spec, node_spec],
        out_shape=[jax.ShapeDtypeStruct((N, D), jnp.float32)] * 2,
    )(x_real, x_imag, *outs, weight, bias.reshape(1, D))
    return out_r, out_i


# speedup vs baseline: 1.0071x; 1.0071x over previous
import functools

import jax
import jax.numpy as jnp
from jax import lax
from jax.experimental import pallas as pl
from jax.experimental.pallas import tpu as pltpu
from jax.experimental.pallas import tpu_sc as plsc

N = 10000
D = 128
DH = D // 2
E = 320000

NC = 2
NT = 16
WIN = 128
NWIN = 159
E_PAD = NT * NWIN * WIN
NROW = 10112
CROWS = NROW // NT


def _sc_mesh_kernel():
    mesh = plsc.VectorSubcoreMesh(core_axis_name="c", subcore_axis_name="s")
    out = jax.ShapeDtypeStruct((NC, NROW, DH), jnp.float32)

    @functools.partial(
        pl.kernel,
        mesh=mesh,
        out_type=[out] * 8,
        compiler_params=pltpu.CompilerParams(use_tc_tiling_on_sc=False),
        scratch_types=[
            pltpu.VMEM((NWIN, WIN), jnp.int32),
            pltpu.VMEM((NWIN, WIN), jnp.int32),
            pltpu.VMEM((NWIN, WIN), jnp.float32),
            pltpu.VMEM((3, WIN, DH), jnp.float32),
            pltpu.VMEM_SHARED((NROW, DH), jnp.float32),
            pltpu.SemaphoreType.DMA,
            pltpu.SemaphoreType.DMA,
            pltpu.SemaphoreType.DMA,
        ],
    )
    def sc_spmm(g00, g01, g10, g11, src_h, dst_h, norm_h, zeros_h,
                o100, o101, o110, o111, o200, o201, o210, o211,
                src_v, dst_v, norm_v, rows3, acc, sga, sgb, sgc):
        c = lax.axis_index("c")
        s = lax.axis_index("s")
        pltpu.sync_copy(src_h.at[s], src_v)
        pltpu.sync_copy(dst_h.at[s], dst_v)
        pltpu.sync_copy(norm_h.at[c].at[s], norm_v)
        o1 = ((o100, o101), (o110, o111))
        o2 = ((o200, o201), (o210, o211))

        bufs = (rows3.at[0], rows3.at[1], rows3.at[2])
        sgs = (sga, sgb, sgc)

        def scale(buf, w):
            @plsc.parallel_loop(0, WIN // 16, unroll=2)
            def _grp(g):
                nvec = norm_v[w, pl.ds(g * 16, 16)]
                for k in range(16):
                    nv = nvec[k]
                    r = g * 16 + k
                    for j in range(DH // 16):
                        sl = pl.ds(j * 16, 16)
                        buf[r, sl] = buf[r, sl] * nv

        for lvl in range(2):
            for p in range(2):
                for h in range(2):
                    oref = (o1 if lvl == 0 else o2)[p][h]
                    if lvl == 0:
                        gt = ((g00, g01), (g10, g11))[p][h]
                    else:
                        gt = o1[p][h].at[c]
                    pltpu.sync_copy(zeros_h.at[pl.ds(s * CROWS, CROWS)],
                                    acc.at[pl.ds(s * CROWS, CROWS)])
                    plsc.subcore_barrier()

                    for b in range(2):
                        pltpu.make_async_copy(gt.at[src_v.at[b]], bufs[b],
                                              sgs[b]).start()

                    def tri_body(t, _):
                        w0 = 3 * t
                        for b in range(3):
                            w = w0 + b
                            nb = (b + 2) % 3
                            pltpu.make_async_copy(gt.at[src_v.at[w]],
                                                  bufs[b], sgs[b]).wait()

                            @pl.when(w + 2 < NWIN)
                            def _():
                                pltpu.make_async_copy(
                                    gt.at[src_v.at[w + 2]], bufs[nb],
                                    sgs[nb]).start()

                            scale(bufs[b], w)
                            pltpu.sync_copy(bufs[b], acc.at[dst_v.at[w]],
                                            add=True)
                        return 0

                    lax.fori_loop(0, NWIN // 3, tri_body, 0, unroll=False)
                    plsc.subcore_barrier()
                    pltpu.sync_copy(acc.at[pl.ds(s * CROWS, CROWS)],
                                    oref.at[c].at[pl.ds(s * CROWS, CROWS)])
                    plsc.subcore_barrier()

    return sc_spmm


_BLK = 1000


def _tc_head(xr, xi, a00, a01, a10, a11, b00, b01, b10, b11, w, b,
             out_r, out_i):
    f32 = jnp.float32
    w02 = w[0] - w[2]
    u_m_v = jnp.concatenate([a00[0] - a10[1], a01[0] - a11[1]], axis=1)
    p_m_q = jnp.concatenate([b00[0] - b10[1], b01[0] - b11[1]], axis=1)
    u_p_v = jnp.concatenate([a10[0] + a00[1], a11[0] + a01[1]], axis=1)
    p_p_q = jnp.concatenate([b10[0] + b00[1], b11[0] + b01[1]], axis=1)
    or_ = jnp.dot(xr[...], w02, preferred_element_type=f32)
    or_ += jnp.dot(u_m_v, w[1], preferred_element_type=f32)
    or_ += 2.0 * jnp.dot(p_m_q, w[2], preferred_element_type=f32)
    oi_ = jnp.dot(xi[...], w02, preferred_element_type=f32)
    oi_ += jnp.dot(u_p_v, w[1], preferred_element_type=f32)
    oi_ += 2.0 * jnp.dot(p_p_q, w[2], preferred_element_type=f32)
    out_r[...] = or_ + b[...]
    out_i[...] = oi_ + b[...]


def kernel(x_real, x_imag, edge_index, norm_real, norm_imag, weight, bias):
    pad = E_PAD - E
    src = jnp.concatenate([edge_index[1], jnp.zeros((pad,), jnp.int32)])
    dst = jnp.concatenate(
        [edge_index[0], N + (jnp.arange(pad, dtype=jnp.int32) % 16)])
    nr = jnp.concatenate([norm_real, jnp.zeros((pad,), jnp.float32)])
    ni = jnp.concatenate([norm_imag, jnp.zeros((pad,), jnp.float32)])

    src3 = src.reshape(NT, NWIN, WIN)
    dst3 = dst.reshape(NT, NWIN, WIN)
    norms = jnp.stack([nr, ni]).reshape(NC, NT, NWIN, WIN)
    g00, g01 = x_real[:, :DH], x_real[:, DH:]
    g10, g11 = x_imag[:, :DH], x_imag[:, DH:]
    zeros = jnp.zeros((NROW, DH), jnp.float32)

    outs = _sc_mesh_kernel()(g00, g01, g10, g11, src3, dst3, norms, zeros)

    nb = pl.cdiv(N, _BLK)
    node_spec = pl.BlockSpec((_BLK, D), lambda i: (i, 0))
    pair_spec = pl.BlockSpec((NC, _BLK, DH), lambda i: (0, i, 0))
    out_r, out_i = pl.pallas_call(
        _tc_head,
        grid=(nb,),
        in_specs=[
            node_spec, node_spec,
            pair_spec, pair_spec, pair_spec, pair_spec,
            pair_spec, pair_spec, pair_spec, pair_spec,
            pl.BlockSpec((3, D, D), lambda i: (0, 0, 0)),
            pl.BlockSpec((1, D), lambda i: (0, 0)),
        ],
        out_specs=[node_spec, node_spec],
        out_shape=[jax.ShapeDtypeStruct((N, D), jnp.float32)] * 2,
    )(x_real, x_imag, *outs, weight, bias.reshape(1, D))
    return out_r, out_i


# speedup vs baseline: 1.1548x; 1.1466x over previous
import functools

import jax
import jax.numpy as jnp
from jax import lax
from jax.experimental import pallas as pl
from jax.experimental.pallas import tpu as pltpu
from jax.experimental.pallas import tpu_sc as plsc

N = 10000
D = 128
DH = D // 2
E = 320000

NC = 2
NT = 16
WIN = 128
NWIN = 158
E_PAD = NT * NWIN * WIN
NROW = 10112
CROWS = NROW // NT


def _sc_mesh_kernel():
    mesh = plsc.VectorSubcoreMesh(core_axis_name="c", subcore_axis_name="s")
    out = jax.ShapeDtypeStruct((NC, NROW, DH), jnp.float32)

    @functools.partial(
        pl.kernel,
        mesh=mesh,
        out_type=[out] * 8,
        compiler_params=pltpu.CompilerParams(use_tc_tiling_on_sc=False),
        scratch_types=[
            pltpu.VMEM((NWIN, WIN), jnp.int32),
            pltpu.VMEM((NWIN, WIN), jnp.int32),
            pltpu.VMEM((NWIN, WIN), jnp.float32),
            pltpu.VMEM((2, WIN, DH), jnp.float32),
            pltpu.VMEM_SHARED((NROW, DH), jnp.float32),
            pltpu.SemaphoreType.DMA,
            pltpu.SemaphoreType.DMA,
        ],
    )
    def sc_spmm(g00, g01, g10, g11, src_h, dst_h, norm_h, zeros_h,
                o100, o101, o110, o111, o200, o201, o210, o211,
                src_v, dst_v, norm_v, rows2, acc, sem_a, sem_b):
        c = lax.axis_index("c")
        s = lax.axis_index("s")
        pltpu.sync_copy(src_h.at[s], src_v)
        pltpu.sync_copy(dst_h.at[s], dst_v)
        pltpu.sync_copy(norm_h.at[c].at[s], norm_v)
        o1 = ((o100, o101), (o110, o111))
        o2 = ((o200, o201), (o210, o211))

        bufs = (rows2.at[0], rows2.at[1])
        sems = (sem_a, sem_b)

        def scale(buf, w):
            @plsc.parallel_loop(0, WIN // 16, unroll=4)
            def _grp(g):
                nvec = norm_v[w, pl.ds(g * 16, 16)]
                for k in range(16):
                    nv = nvec[k]
                    r = g * 16 + k
                    for j in range(DH // 16):
                        sl = pl.ds(j * 16, 16)
                        buf[r, sl] = buf[r, sl] * nv

        for lvl in range(2):
            for p in range(2):
                for h in range(2):
                    oref = (o1 if lvl == 0 else o2)[p][h]
                    if lvl == 0:
                        gt = ((g00, g01), (g10, g11))[p][h]
                    else:
                        gt = o1[p][h].at[c]
                    pltpu.sync_copy(zeros_h.at[pl.ds(s * CROWS, CROWS)],
                                    acc.at[pl.ds(s * CROWS, CROWS)])
                    plsc.subcore_barrier()

                    for b in range(2):
                        pltpu.make_async_copy(gt.at[src_v.at[b]], bufs[b],
                                              sems[b]).start()

                    def pair_body(t, _):
                        w0 = 2 * t
                        for b in range(2):
                            w = w0 + b
                            buf, sem = bufs[b], sems[b]
                            pltpu.make_async_copy(gt.at[src_v.at[w]], buf,
                                                  sem).wait()
                            scale(buf, w)
                            pltpu.sync_copy(buf, acc.at[dst_v.at[w]],
                                            add=True)

                            @pl.when(w + 2 < NWIN)
                            def _():
                                pltpu.make_async_copy(
                                    gt.at[src_v.at[w + 2]], buf,
                                    sem).start()
                        return 0

                    lax.fori_loop(0, NWIN // 2, pair_body, 0, unroll=False)
                    plsc.subcore_barrier()
                    pltpu.sync_copy(acc.at[pl.ds(s * CROWS, CROWS)],
                                    oref.at[c].at[pl.ds(s * CROWS, CROWS)])
                    plsc.subcore_barrier()

    return sc_spmm


_BLK = 1000


def _tc_head(xr, xi, a00, a01, a10, a11, b00, b01, b10, b11, w, b,
             out_r, out_i):
    f32 = jnp.float32
    w02 = w[0] - w[2]
    u_m_v = jnp.concatenate([a00[0] - a10[1], a01[0] - a11[1]], axis=1)
    p_m_q = jnp.concatenate([b00[0] - b10[1], b01[0] - b11[1]], axis=1)
    u_p_v = jnp.concatenate([a10[0] + a00[1], a11[0] + a01[1]], axis=1)
    p_p_q = jnp.concatenate([b10[0] + b00[1], b11[0] + b01[1]], axis=1)
    or_ = jnp.dot(xr[...], w02, preferred_element_type=f32)
    or_ += jnp.dot(u_m_v, w[1], preferred_element_type=f32)
    or_ += 2.0 * jnp.dot(p_m_q, w[2], preferred_element_type=f32)
    oi_ = jnp.dot(xi[...], w02, preferred_element_type=f32)
    oi_ += jnp.dot(u_p_v, w[1], preferred_element_type=f32)
    oi_ += 2.0 * jnp.dot(p_p_q, w[2], preferred_element_type=f32)
    out_r[...] = or_ + b[...]
    out_i[...] = oi_ + b[...]


def kernel(x_real, x_imag, edge_index, norm_real, norm_imag, weight, bias):
    pad = E_PAD - E
    src = jnp.concatenate([edge_index[1], jnp.zeros((pad,), jnp.int32)])
    dst = jnp.concatenate(
        [edge_index[0], N + (jnp.arange(pad, dtype=jnp.int32) % 16)])
    nr = jnp.concatenate([norm_real, jnp.zeros((pad,), jnp.float32)])
    ni = jnp.concatenate([norm_imag, jnp.zeros((pad,), jnp.float32)])

    src3 = src.reshape(NT, NWIN, WIN)
    dst3 = dst.reshape(NT, NWIN, WIN)
    norms = jnp.stack([nr, ni]).reshape(NC, NT, NWIN, WIN)
    g00, g01 = x_real[:, :DH], x_real[:, DH:]
    g10, g11 = x_imag[:, :DH], x_imag[:, DH:]
    zeros = jnp.zeros((NROW, DH), jnp.float32)

    outs = _sc_mesh_kernel()(g00, g01, g10, g11, src3, dst3, norms, zeros)

    nb = pl.cdiv(N, _BLK)
    node_spec = pl.BlockSpec((_BLK, D), lambda i: (i, 0))
    pair_spec = pl.BlockSpec((NC, _BLK, DH), lambda i: (0, i, 0))
    out_r, out_i = pl.pallas_call(
        _tc_head,
        grid=(nb,),
        in_specs=[
            node_spec, node_spec,
            pair_spec, pair_spec, pair_spec, pair_spec,
            pair_spec, pair_spec, pair_spec, pair_spec,
            pl.BlockSpec((3, D, D), lambda i: (0, 0, 0)),
            pl.BlockSpec((1, D), lambda i: (0, 0)),
        ],
        out_specs=[node_spec, node_spec],
        out_shape=[jax.ShapeDtypeStruct((N, D), jnp.float32)] * 2,
    )(x_real, x_imag, *outs, weight, bias.reshape(1, D))
    return out_r, out_i


# speedup vs baseline: 1.5566x; 1.3479x over previous
import functools

import jax
import jax.numpy as jnp
from jax import lax
from jax.experimental import pallas as pl
from jax.experimental.pallas import tpu as pltpu
from jax.experimental.pallas import tpu_sc as plsc

N = 10000
D = 128
DH = D // 2
E = 320000

NC = 2
NT = 16
WIN = 128
NWIN = 158
E_PAD = NT * NWIN * WIN
NROW = 10112
CROWS = NROW // NT


def _sc_mesh_kernel():
    mesh = plsc.VectorSubcoreMesh(core_axis_name="c", subcore_axis_name="s")
    out = jax.ShapeDtypeStruct((NC, NROW, DH), jnp.float32)
    outb = jax.ShapeDtypeStruct((NC, NROW, DH), jnp.bfloat16)

    @functools.partial(
        pl.kernel,
        mesh=mesh,
        out_type=[out] * 8 + [outb] * 4,
        compiler_params=pltpu.CompilerParams(use_tc_tiling_on_sc=False,
                                            needs_layout_passes=False),
        scratch_types=[
            pltpu.VMEM((NWIN, WIN), jnp.int32),
            pltpu.VMEM((NWIN, WIN), jnp.int32),
            pltpu.VMEM((NWIN, WIN), jnp.float32),
            pltpu.VMEM((2, WIN, DH), jnp.bfloat16),
            pltpu.VMEM((WIN, DH), jnp.float32),
            pltpu.VMEM_SHARED((NROW, DH), jnp.float32),
            pltpu.SemaphoreType.DMA,
            pltpu.SemaphoreType.DMA,
        ],
    )
    def sc_spmm(g00, g01, g10, g11, src_h, dst_h, norm_h, zeros_h,
                o100, o101, o110, o111, o200, o201, o210, o211,
                b100, b101, b110, b111,
                src_v, dst_v, norm_v, rows2, scaled, acc, sem_a, sem_b):
        c = lax.axis_index("c")
        s = lax.axis_index("s")
        pltpu.sync_copy(src_h.at[s], src_v)
        pltpu.sync_copy(dst_h.at[s], dst_v)
        pltpu.sync_copy(norm_h.at[c].at[s], norm_v)
        o1 = ((o100, o101), (o110, o111))
        o2 = ((o200, o201), (o210, o211))
        o1b = ((b100, b101), (b110, b111))

        bufs = (rows2.at[0], rows2.at[1])
        sems = (sem_a, sem_b)

        def scale(buf, w):
            @plsc.parallel_loop(0, WIN // 16, unroll=4)
            def _grp(g):
                nvec = norm_v[w, pl.ds(g * 16, 16)]
                for k in range(16):
                    nv = nvec[k]
                    r = g * 16 + k
                    for j in range(DH // 32):
                        v32 = buf[r, pl.ds(j * 32, 32)]
                        va, vb = plsc.unpack(
                            v32, format=plsc.PackFormat.INTERLEAVED)
                        scaled[r, pl.ds(j * 32, 16)] = va * nv
                        scaled[r, pl.ds(j * 32 + 16, 16)] = vb * nv

        for lvl in range(2):
            for p in range(2):
                for h in range(2):
                    oref = (o1 if lvl == 0 else o2)[p][h]
                    if lvl == 0:
                        gt = ((g00, g01), (g10, g11))[p][h]
                    else:
                        gt = o1b[p][h].at[c]
                    pltpu.sync_copy(zeros_h.at[pl.ds(s * CROWS, CROWS)],
                                    acc.at[pl.ds(s * CROWS, CROWS)])
                    plsc.subcore_barrier()

                    for b in range(2):
                        pltpu.make_async_copy(gt.at[src_v.at[b]], bufs[b],
                                              sems[b]).start()

                    def pair_body(t, _):
                        w0 = 2 * t
                        for b in range(2):
                            w = w0 + b
                            buf, sem = bufs[b], sems[b]
                            pltpu.make_async_copy(gt.at[src_v.at[w]], buf,
                                                  sem).wait()
                            scale(buf, w)
                            pltpu.sync_copy(scaled, acc.at[dst_v.at[w]],
                                            add=True)

                            @pl.when(w + 2 < NWIN)
                            def _():
                                pltpu.make_async_copy(
                                    gt.at[src_v.at[w + 2]], buf,
                                    sem).start()
                        return 0

                    lax.fori_loop(0, NWIN // 2, pair_body, 0, unroll=False)
                    plsc.subcore_barrier()
                    pltpu.sync_copy(acc.at[pl.ds(s * CROWS, CROWS)],
                                    oref.at[c].at[pl.ds(s * CROWS, CROWS)])
                    if lvl == 0:
                        bref = o1b[p][h]
                        cb = CROWS // 8

                        def conv_chunk(q, _):
                            r0 = s * CROWS + q * cb
                            pltpu.sync_copy(acc.at[pl.ds(r0, cb)],
                                            scaled.at[pl.ds(0, cb)])

                            @plsc.parallel_loop(0, cb, unroll=2)
                            def _row(r):
                                for j in range(DH // 32):
                                    va = scaled[r, pl.ds(j * 32, 16)]
                                    vb = scaled[r, pl.ds(j * 32 + 16, 16)]
                                    rows2[0, r, pl.ds(j * 32, 32)] = (
                                        plsc.pack(
                                            va, vb,
                                            format=plsc.PackFormat
                                            .INTERLEAVED))

                            pltpu.sync_copy(
                                rows2.at[0].at[pl.ds(0, cb)],
                                bref.at[c].at[pl.ds(r0, cb)])
                            return 0

                        lax.fori_loop(0, 8, conv_chunk, 0, unroll=False)
                    plsc.subcore_barrier()

    return sc_spmm


_BLK = 1000


def _tc_head(xr, xi, a00, a01, a10, a11, b00, b01, b10, b11, w, b,
             out_r, out_i):
    f32 = jnp.float32
    w02 = w[0] - w[2]
    u_m_v = jnp.concatenate([a00[0] - a10[1], a01[0] - a11[1]], axis=1)
    p_m_q = jnp.concatenate([b00[0] - b10[1], b01[0] - b11[1]], axis=1)
    u_p_v = jnp.concatenate([a10[0] + a00[1], a11[0] + a01[1]], axis=1)
    p_p_q = jnp.concatenate([b10[0] + b00[1], b11[0] + b01[1]], axis=1)
    or_ = jnp.dot(xr[...], w02, preferred_element_type=f32)
    or_ += jnp.dot(u_m_v, w[1], preferred_element_type=f32)
    or_ += 2.0 * jnp.dot(p_m_q, w[2], preferred_element_type=f32)
    oi_ = jnp.dot(xi[...], w02, preferred_element_type=f32)
    oi_ += jnp.dot(u_p_v, w[1], preferred_element_type=f32)
    oi_ += 2.0 * jnp.dot(p_p_q, w[2], preferred_element_type=f32)
    out_r[...] = or_ + b[...]
    out_i[...] = oi_ + b[...]


def kernel(x_real, x_imag, edge_index, norm_real, norm_imag, weight, bias):
    pad = E_PAD - E
    src = jnp.concatenate([edge_index[1], jnp.zeros((pad,), jnp.int32)])
    dst = jnp.concatenate(
        [edge_index[0], N + (jnp.arange(pad, dtype=jnp.int32) % 16)])
    nr = jnp.concatenate([norm_real, jnp.zeros((pad,), jnp.float32)])
    ni = jnp.concatenate([norm_imag, jnp.zeros((pad,), jnp.float32)])

    src3 = src.reshape(NT, NWIN, WIN)
    dst3 = dst.reshape(NT, NWIN, WIN)
    norms = jnp.stack([nr, ni]).reshape(NC, NT, NWIN, WIN)
    def b16i(x):
        x4 = x.reshape(N, DH // 32, 2, 16)
        return x4.transpose(0, 1, 3, 2).reshape(N, DH).astype(jnp.bfloat16)

    g00, g01 = b16i(x_real[:, :DH]), b16i(x_real[:, DH:])
    g10, g11 = b16i(x_imag[:, :DH]), b16i(x_imag[:, DH:])
    zeros = jnp.zeros((NROW, DH), jnp.float32)

    outs = _sc_mesh_kernel()(g00, g01, g10, g11, src3, dst3, norms,
                             zeros)[:8]

    nb = pl.cdiv(N, _BLK)
    node_spec = pl.BlockSpec((_BLK, D), lambda i: (i, 0))
    pair_spec = pl.BlockSpec((NC, _BLK, DH), lambda i: (0, i, 0))
    out_r, out_i = pl.pallas_call(
        _tc_head,
        grid=(nb,),
        in_specs=[
            node_spec, node_spec,
            pair_spec, pair_spec, pair_spec, pair_spec,
            pair_spec, pair_spec, pair_spec, pair_spec,
            pl.BlockSpec((3, D, D), lambda i: (0, 0, 0)),
            pl.BlockSpec((1, D), lambda i: (0, 0)),
        ],
        out_specs=[node_spec, node_spec],
        out_shape=[jax.ShapeDtypeStruct((N, D), jnp.float32)] * 2,
    )(x_real, x_imag, *outs, weight, bias.reshape(1, D))
    return out_r, out_i
